# baseline (speedup 1.0000x reference)
import jax
import jax.numpy as jnp
from jax.experimental import pallas as pl
from jax.experimental.pallas import tpu as pltpu

_DROP_PROB = 0.1
_ROWS_PER_BLOCK = 256


def _drop_body(rand_ref, pad_ref, x_ref, out_ref):
    mask = rand_ref[...] < _DROP_PROB
    out_ref[...] = jnp.where(mask, pad_ref[...], x_ref[...])


def kernel(x, rand_tensor, pad):
    input_shape = x.shape
    dim = input_shape[-1]
    x_flat = jnp.reshape(x, (-1, dim))
    n_rows = x_flat.shape[0]
    rand2d = jnp.reshape(rand_tensor, (n_rows, 1))
    pad2d = jnp.reshape(pad, (1, dim))
    blk = _ROWS_PER_BLOCK
    grid = (n_rows // blk,)
    out = pl.pallas_call(
        _drop_body,
        grid=grid,
        in_specs=[
            pl.BlockSpec((blk, 1), lambda i: (i, 0)),
            pl.BlockSpec((1, dim), lambda i: (0, 0)),
            pl.BlockSpec((blk, dim), lambda i: (i, 0)),
        ],
        out_specs=pl.BlockSpec((blk, dim), lambda i: (i, 0)),
        out_shape=jax.ShapeDtypeStruct((n_rows, dim), x.dtype),
        compiler_params=pltpu.CompilerParams(
            dimension_semantics=("arbitrary",),
        ),
    )(rand2d, pad2d, x_flat)
    return jnp.reshape(out, input_shape)


# speedup vs baseline: 1.4633x; 1.4633x over previous
import jax
import jax.numpy as jnp
from jax.experimental import pallas as pl
from jax.experimental.pallas import tpu as pltpu

_DROP_PROB = 0.1
_ROWS_PER_BLOCK = 2048


def _drop_body(rand_ref, pad_ref, x_ref, out_ref):
    mask = rand_ref[...] < _DROP_PROB
    out_ref[...] = jnp.where(mask, pad_ref[...], x_ref[...])


def kernel(x, rand_tensor, pad):
    input_shape = x.shape
    dim = input_shape[-1]
    x_flat = jnp.reshape(x, (-1, dim))
    n_rows = x_flat.shape[0]
    rand2d = jnp.reshape(rand_tensor, (n_rows, 1))
    pad2d = jnp.reshape(pad, (1, dim))
    blk = _ROWS_PER_BLOCK
    grid = (n_rows // blk,)
    out = pl.pallas_call(
        _drop_body,
        grid=grid,
        in_specs=[
            pl.BlockSpec((blk, 1), lambda i: (i, 0)),
            pl.BlockSpec((1, dim), lambda i: (0, 0)),
            pl.BlockSpec((blk, dim), lambda i: (i, 0)),
        ],
        out_specs=pl.BlockSpec((blk, dim), lambda i: (i, 0)),
        out_shape=jax.ShapeDtypeStruct((n_rows, dim), x.dtype),
        compiler_params=pltpu.CompilerParams(
            dimension_semantics=("arbitrary",),
        ),
    )(rand2d, pad2d, x_flat)
    return jnp.reshape(out, input_shape)
